# baseline (speedup 1.0000x reference)
import jax
import jax.numpy as jnp
from jax import lax
from jax.experimental import pallas as pl
from jax.experimental.pallas import tpu as pltpu
from jax.experimental.pallas import tpu_sc as plsc

_L = 20
_BATCH = 16384
_NC, _NS, _LANES = 2, 16, 16
_NW = _NC * _NS
_BPW = _BATCH // _NW
_GCH = 128
_NG = _BPW // _GCH
_VPG = _GCH // _LANES
_TCB = 4096

_LN2 = 0.6931471805599453
_LOGP = (
    0.010119082927824848,
    -0.052624851367851076,
    0.13076503250423846,
    -0.2228362583280196,
    0.32697310001386687,
    -0.4992065685478449,
    0.9999574870750662,
    5.621959008883515e-07,
)


def _pack_tc_body(xt_ref, idx_ref):
    accs = [None] * 4
    for i in range(_L):
        z = xt_ref[i, :] << (_L - 1 - i)
        k = i & 3
        accs[k] = z if accs[k] is None else accs[k] + z
    acc = (accs[0] + accs[1]) + (accs[2] + accs[3])
    idx_ref[...] = (acc + ((1 << _L) - 1)) >> 1


def _log16(a):
    bits = lax.bitcast_convert_type(a, jnp.int32)
    e = ((bits >> 23) - 127).astype(jnp.float32)
    m = lax.bitcast_convert_type(
        (bits & 0x007FFFFF) | 0x3F800000, jnp.float32)
    t = m - 1.0
    p = jnp.float32(_LOGP[0])
    for c in _LOGP[1:]:
        p = p * t + jnp.float32(c)
    return e * jnp.float32(_LN2) + p


def _ewf_sc_body(idx_hbm, aux_hbm, out_hbm, idxv, av, gsem):
    wid = lax.axis_index("s") * _NC + lax.axis_index("c")
    base = pl.multiple_of(wid * _BPW, _BPW)

    pltpu.sync_copy(idx_hbm.at[pl.ds(base, _BPW)], idxv)
    pltpu.async_copy(aux_hbm.at[idxv], av, gsem).wait()
    for v in range(_BPW // _LANES):
        sl = pl.ds(v * _LANES, _LANES)
        av[sl] = _log16(av[sl])
    pltpu.sync_copy(av, out_hbm.at[pl.ds(base, _BPW)])


@jax.jit
def _ewf(x, aux):
    xt = jnp.transpose(x)
    idx = pl.pallas_call(
        _pack_tc_body,
        grid=(_BATCH // _TCB,),
        in_specs=[pl.BlockSpec((_L, _TCB), lambda i: (0, i))],
        out_specs=pl.BlockSpec((_TCB,), lambda i: (i,)),
        out_shape=jax.ShapeDtypeStruct((_BATCH,), jnp.int32),
    )(xt)

    mesh = plsc.VectorSubcoreMesh(core_axis_name="c", subcore_axis_name="s")
    return pl.kernel(
        _ewf_sc_body,
        out_type=jax.ShapeDtypeStruct((_BATCH,), jnp.float32),
        mesh=mesh,
        scratch_types=[
            pltpu.VMEM((_BPW,), jnp.int32),
            pltpu.VMEM((_BPW,), jnp.float32),
            pltpu.SemaphoreType.DMA,
        ],
    )(idx, aux)


def kernel(x, aux):
    return _ewf(x, aux)


# speedup vs baseline: 1.0489x; 1.0489x over previous
import jax
import jax.numpy as jnp
from jax import lax
from jax.experimental import pallas as pl
from jax.experimental.pallas import tpu as pltpu
from jax.experimental.pallas import tpu_sc as plsc

_L = 20
_BATCH = 16384
_NC, _NS, _LANES = 2, 16, 16
_NW = _NC * _NS
_BPW = _BATCH // _NW
_GCH = 128
_NG = _BPW // _GCH
_VPG = _GCH // _LANES
_TCB = 4096

_LN2 = 0.6931471805599453
_LOGP = (
    0.010119082927824848,
    -0.052624851367851076,
    0.13076503250423846,
    -0.2228362583280196,
    0.32697310001386687,
    -0.4992065685478449,
    0.9999574870750662,
    5.621959008883515e-07,
)


def _pack_tc_body(xt_ref, idx_ref):
    accs = [None] * 4
    for i in range(_L):
        z = xt_ref[i, :] << (_L - 1 - i)
        k = i & 3
        accs[k] = z if accs[k] is None else accs[k] + z
    acc = (accs[0] + accs[1]) + (accs[2] + accs[3])
    idx_ref[...] = (acc + ((1 << _L) - 1)) >> 1


def _log16(a):
    bits = lax.bitcast_convert_type(a, jnp.int32)
    e = ((bits >> 23) - 127).astype(jnp.float32)
    m = lax.bitcast_convert_type(
        (bits & 0x007FFFFF) | 0x3F800000, jnp.float32)
    t = m - 1.0
    p = jnp.float32(_LOGP[0])
    for c in _LOGP[1:]:
        p = p * t + jnp.float32(c)
    return e * jnp.float32(_LN2) + p


def _ewf_sc_body(idx_hbm, aux_hbm, out_hbm, idxv, av, gsem):
    wid = lax.axis_index("s") * _NC + lax.axis_index("c")
    base = pl.multiple_of(wid * _BPW, _BPW)

    pltpu.sync_copy(idx_hbm.at[pl.ds(base, _BPW)], idxv)
    pltpu.async_copy(aux_hbm.at[idxv], av, gsem).wait()

    def _log_step(v, carry):
        sl = pl.ds(pl.multiple_of(v * _LANES, _LANES), _LANES)
        av[sl] = _log16(av[sl])
        return carry

    lax.fori_loop(0, _BPW // _LANES, _log_step, 0, unroll=4)
    pltpu.sync_copy(av, out_hbm.at[pl.ds(base, _BPW)])


@jax.jit
def _ewf(x, aux):
    xt = jnp.transpose(x)
    idx = pl.pallas_call(
        _pack_tc_body,
        out_shape=jax.ShapeDtypeStruct((_BATCH,), jnp.int32),
    )(xt)

    mesh = plsc.VectorSubcoreMesh(core_axis_name="c", subcore_axis_name="s")
    return pl.kernel(
        _ewf_sc_body,
        out_type=jax.ShapeDtypeStruct((_BATCH,), jnp.float32),
        mesh=mesh,
        scratch_types=[
            pltpu.VMEM((_BPW,), jnp.int32),
            pltpu.VMEM((_BPW,), jnp.float32),
            pltpu.SemaphoreType.DMA,
        ],
    )(idx, aux)


def kernel(x, aux):
    return _ewf(x, aux)
